# Initial kernel scaffold; baseline (speedup 1.0000x reference)
import jax
import jax.numpy as jnp
from jax.experimental import pallas as pl


def kernel(input_sentence, weight):
    raise NotImplementedError("write your pallas kernel here")


# speedup vs baseline: 4.6329x; 4.6329x over previous
import functools

import jax
import jax.numpy as jnp
from jax import lax
from jax.experimental import pallas as pl
from jax.experimental.pallas import tpu as pltpu
from jax.experimental.pallas import tpu_sc as plsc

VOCAB = 100000
EMBED_DIM = 64
NUM_INDICES = 4096 * 50

_info = plsc.get_sparse_core_info()
NC, NS = _info.num_cores, _info.num_subcores
NW = NC * NS
PER_W = NUM_INDICES // NW
CHUNK = 800
NCHUNK = PER_W // CHUNK


def _embed_kernel(idx_hbm, table_hbm, out_hbm,
                  idx_a, idx_b, rows_a, rows_b, sem_a, sem_b):
    wid = lax.axis_index("s") * NC + lax.axis_index("c")
    base = wid * PER_W

    idx_bufs = (idx_a, idx_b)
    row_bufs = (rows_a, rows_b)
    sems = (sem_a, sem_b)

    pltpu.sync_copy(idx_hbm.at[pl.ds(base, CHUNK)], idx_bufs[0])
    copies = [pltpu.async_copy(table_hbm.at[idx_bufs[0]], row_bufs[0], sems[0])]

    for g in range(NCHUNK):
        b = g % 2
        nb = (g + 1) % 2
        if g + 1 < NCHUNK:
            off = base + (g + 1) * CHUNK
            pltpu.sync_copy(idx_hbm.at[pl.ds(off, CHUNK)], idx_bufs[nb])
            copies.append(
                pltpu.async_copy(table_hbm.at[idx_bufs[nb]], row_bufs[nb],
                                 sems[nb]))
        copies[g].wait()
        pltpu.sync_copy(row_bufs[b], out_hbm.at[pl.ds(base + g * CHUNK, CHUNK)])


@jax.jit
def _embed(idx_flat, weight):
    mesh = plsc.VectorSubcoreMesh(core_axis_name="c", subcore_axis_name="s")
    return pl.kernel(
        _embed_kernel,
        out_type=jax.ShapeDtypeStruct((NUM_INDICES, EMBED_DIM), jnp.float32),
        mesh=mesh,
        scratch_types=[
            pltpu.VMEM((CHUNK,), jnp.int32),
            pltpu.VMEM((CHUNK,), jnp.int32),
            pltpu.VMEM((CHUNK, EMBED_DIM), jnp.float32),
            pltpu.VMEM((CHUNK, EMBED_DIM), jnp.float32),
            pltpu.SemaphoreType.DMA,
            pltpu.SemaphoreType.DMA,
        ],
        compiler_params=pltpu.CompilerParams(use_tc_tiling_on_sc=False),
    )(idx_flat, weight)


def kernel(input_sentence, weight):
    B, S = input_sentence.shape
    idx_flat = input_sentence.reshape(-1).astype(jnp.int32)
    out = _embed(idx_flat, weight)
    return out.reshape(B, S, EMBED_DIM)
